# baseline (speedup 1.0000x reference)
import functools

import jax
import jax.numpy as jnp
from jax import lax
from jax.experimental import pallas as pl
from jax.experimental.pallas import tpu as pltpu
from jax.experimental.pallas import tpu_sc as plsc

NODES = 10000
EDGES = 320000
NROWS = 10240
NC, NS = 2, 16
NW = NC * NS
RZ = NROWS // NS
CH = 128
EW = EDGES // NW
NCH = 80
EWP = NCH * CH
RB = 1024
GR = NROWS // RB


def _mesh():
    return plsc.VectorSubcoreMesh(
        core_axis_name="c", subcore_axis_name="s", num_cores=NC, num_subcores=NS
    )


@functools.partial(
    pl.kernel,
    out_type=jax.ShapeDtypeStruct((NC, NROWS), jnp.float32),
    mesh=_mesh(),
    compiler_params=pltpu.CompilerParams(needs_layout_passes=False, use_tc_tiling_on_sc=False),
    scratch_types=[
        pltpu.VMEM((NCH, CH), jnp.int32),
        pltpu.VMEM((NCH, CH), jnp.float32),
        pltpu.VMEM_SHARED((NROWS,), jnp.float32),
    ],
)
def _deg_kernel(row_hbm, attr_hbm, zeros_hbm, out_hbm, row_v, attr_v, acc):
    c = lax.axis_index("c")
    s = lax.axis_index("s")
    wid = s * NC + c
    pltpu.sync_copy(row_hbm.at[wid], row_v)
    pltpu.sync_copy(attr_hbm.at[wid], attr_v)
    pltpu.sync_copy(zeros_hbm.at[pl.ds(s * RZ, RZ)], acc.at[pl.ds(s * RZ, RZ)])
    plsc.subcore_barrier()

    def body(j, carry):
        pltpu.sync_copy(attr_v.at[j], acc.at[row_v.at[j]], add=True)
        return carry

    lax.fori_loop(0, NCH, body, 0)
    plsc.subcore_barrier()
    pltpu.sync_copy(acc.at[pl.ds(s * RZ, RZ)], out_hbm.at[c, pl.ds(s * RZ, RZ)])


@functools.partial(
    pl.kernel,
    out_type=jax.ShapeDtypeStruct((NW, NCH, CH), jnp.float32),
    mesh=_mesh(),
    compiler_params=pltpu.CompilerParams(needs_layout_passes=False, use_tc_tiling_on_sc=False),
    scratch_types=[
        pltpu.VMEM((NCH, CH), jnp.int32),
        pltpu.VMEM((NCH, CH), jnp.int32),
        pltpu.VMEM((NCH, CH), jnp.float32),
        pltpu.VMEM((NCH, CH), jnp.float32),
        pltpu.VMEM((NROWS,), jnp.float32),
    ],
)
def _norm_kernel(row_hbm, col_hbm, attr_hbm, dinv_hbm, out_hbm,
                 row_v, col_v, attr_v, w_v, dinv_v):
    c = lax.axis_index("c")
    s = lax.axis_index("s")
    wid = s * NC + c
    pltpu.sync_copy(row_hbm.at[wid], row_v)
    pltpu.sync_copy(col_hbm.at[wid], col_v)
    pltpu.sync_copy(attr_hbm.at[wid], attr_v)
    pltpu.sync_copy(dinv_hbm, dinv_v)

    def body(j, carry):
        for g in range(CH // 16):
            sl = pl.ds(g * 16, 16)
            r16 = row_v[j, sl]
            c16 = col_v[j, sl]
            a16 = attr_v[j, sl]
            dr = plsc.load_gather(dinv_v, [r16])
            dc = plsc.load_gather(dinv_v, [c16])
            w_v[j, sl] = -(dr * a16 * dc)
        return carry

    lax.fori_loop(0, NCH, body, 0)
    pltpu.sync_copy(w_v, out_hbm.at[wid])


def _make_prop(F):
    nq = F // 16
    NB = 4

    @functools.partial(
        pl.kernel,
        out_type=jax.ShapeDtypeStruct((NC, NROWS, F), jnp.float32),
        mesh=_mesh(),
        compiler_params=pltpu.CompilerParams(needs_layout_passes=False, use_tc_tiling_on_sc=False),
        scratch_types=[
            pltpu.VMEM((NCH, CH), jnp.int32),
            pltpu.VMEM((NCH, CH), jnp.int32),
            pltpu.VMEM((NCH, CH), jnp.float32),
            pltpu.VMEM((NB, CH, F), jnp.float32),
            pltpu.VMEM((16, F), jnp.float32),
            pltpu.VMEM_SHARED((NROWS, F), jnp.float32),
        ] + [pltpu.SemaphoreType.DMA] * (2 * NB),
    )
    def prop(table_hbm, col_hbm, row_hbm, w_hbm, out_hbm,
             col_v, row_v, w_v, rbuf, zbuf, acc, *sems):
        gsems = sems[:NB]
        ssems = sems[NB:]
        c = lax.axis_index("c")
        s = lax.axis_index("s")
        wid = s * NC + c
        pltpu.sync_copy(col_hbm.at[wid], col_v)
        pltpu.sync_copy(row_hbm.at[wid], row_v)
        pltpu.sync_copy(w_hbm.at[wid], w_v)

        zv = jnp.zeros((16,), jnp.float32)

        def zrow(r, carry):
            for q in range(nq):
                zbuf[r, pl.ds(q * 16, 16)] = zv
            return carry

        lax.fori_loop(0, 16, zrow, 0)
        for i in range(RZ // 16):
            pltpu.sync_copy(zbuf, acc.at[pl.ds(s * RZ + i * 16, 16)])
        plsc.subcore_barrier()

        def g_start(j, b):
            pltpu.async_copy(table_hbm.at[col_v.at[j]], rbuf.at[b], gsems[b])

        def g_wait(j, b):
            pltpu.make_async_copy(table_hbm.at[col_v.at[j]], rbuf.at[b],
                                  gsems[b]).wait()

        def s_start(j, b):
            pltpu.async_copy(rbuf.at[b], acc.at[row_v.at[j]], ssems[b],
                             add=True)

        def s_wait(j, b):
            pltpu.make_async_copy(rbuf.at[b], acc.at[row_v.at[j]],
                                  ssems[b]).wait()

        g_start(0, 0)
        g_start(1, 1)

        def quad(j4, carry):
            for b in range(NB):
                j = j4 * NB + b
                g_wait(j, b)

                def scale(g, c2):
                    w16 = w_v[j, pl.ds(g * 16, 16)]
                    for u in range(16):
                        e = g * 16 + u
                        we = w16[u]
                        for q in range(nq):
                            sl = pl.ds(q * 16, 16)
                            rbuf[b, e, sl] = rbuf[b, e, sl] * we
                    return c2

                lax.fori_loop(0, CH // 16, scale, 0)
                s_start(j, b)
                bn = (b + 2) % NB

                @pl.when(j >= 2)
                def _():
                    s_wait(j - 2, bn)

                @pl.when(j + 2 < NCH)
                def _():
                    g_start(j + 2, bn)

            return carry

        lax.fori_loop(0, NCH // NB, quad, 0)
        s_wait(NCH - 2, (NCH - 2) % NB)
        s_wait(NCH - 1, (NCH - 1) % NB)
        plsc.subcore_barrier()
        pltpu.sync_copy(acc.at[pl.ds(s * RZ, RZ)],
                        out_hbm.at[c, pl.ds(s * RZ, RZ)])

    return prop


_PROPS = {F: _make_prop(F) for F in (64, 32, 16)}


@functools.partial(
    pl.kernel,
    out_type=jax.ShapeDtypeStruct((NW, NROWS * 4), jnp.float32),
    mesh=_mesh(),
    compiler_params=pltpu.CompilerParams(needs_layout_passes=False, use_tc_tiling_on_sc=False),
    scratch_types=[
        pltpu.VMEM((NCH, CH), jnp.int32),
        pltpu.VMEM((NCH, CH), jnp.int32),
        pltpu.VMEM((NCH, CH), jnp.float32),
        pltpu.VMEM((NROWS * 4,), jnp.float32),
        pltpu.VMEM((NROWS * 4,), jnp.float32),
    ],
)
def _prop4(table_hbm, col_hbm, row_hbm, w_hbm, out_hbm,
           col_v, row_v, w_v, tbl_v, acc_v):
    c = lax.axis_index("c")
    s = lax.axis_index("s")
    wid = s * NC + c
    pltpu.sync_copy(col_hbm.at[wid], col_v)
    pltpu.sync_copy(row_hbm.at[wid], row_v)
    pltpu.sync_copy(w_hbm.at[wid], w_v)
    pltpu.sync_copy(table_hbm, tbl_v)

    zv = jnp.zeros((16,), jnp.float32)

    def zrow(r, carry):
        for q in range(4):
            acc_v[pl.ds((r * 4 + q) * 16, 16)] = zv
        return carry

    lax.fori_loop(0, NROWS // 16, zrow, 0)

    def chunk(j, carry):
        for g in range(CH // 16):
            sl = pl.ds(g * 16, 16)
            col4 = col_v[j, sl] * 4
            row4 = row_v[j, sl] * 4
            w16 = w_v[j, sl]
            for f in range(4):
                vals = plsc.load_gather(tbl_v, [col4 + f]) * w16
                plsc.addupdate_scatter(acc_v, [row4 + f], vals)
        return carry

    lax.fori_loop(0, NCH, chunk, 0)
    pltpu.sync_copy(acc_v, out_hbm.at[wid])


def _dinv_body(d_ref, dinv_ref):
    d = d_ref[0] + d_ref[1]
    safe = jnp.where(d > 0, d, 1.0)
    dinv_ref[...] = jnp.where(d > 0, lax.rsqrt(safe), 0.0)


def _dinv_tc(deg2):
    return pl.pallas_call(
        _dinv_body,
        out_shape=jax.ShapeDtypeStruct((NROWS // 128, 128), jnp.float32),
    )(deg2.reshape(NC, NROWS // 128, 128)).reshape(NROWS)


def _make_tca(fi, fo, f2, first):
    def body(*refs):
        if first:
            x_ref, w_ref, b_ref, ybase_ref, y1_ref, y2_ref = refs
            xb = x_ref[...]
        else:
            yb_ref, u_ref, w_ref, b_ref, ybase_ref, y1_ref, y2_ref = refs
            xb = jax.nn.relu(
                yb_ref[...] + (u_ref[0] + u_ref[1])[:, :fi]
            )
        w = w_ref[...]
        ybase_ref[...] = (
            jnp.dot(xb, w[0] - w[2], preferred_element_type=jnp.float32)
            + b_ref[...]
        )
        y1_ref[...] = jnp.dot(xb, w[1], preferred_element_type=jnp.float32)
        y2 = jnp.dot(xb, w[2], preferred_element_type=jnp.float32)
        if f2 > fo:
            y2 = jnp.pad(y2, ((0, 0), (0, f2 - fo)))
        y2_ref[...] = y2

    if first:
        in_specs = [
            pl.BlockSpec((RB, fi), lambda i: (i, 0)),
            pl.BlockSpec((3, fi, fo), lambda i: (0, 0, 0)),
            pl.BlockSpec((1, fo), lambda i: (0, 0)),
        ]
    else:
        fprev = max(fi, 16)
        in_specs = [
            pl.BlockSpec((RB, fi), lambda i: (i, 0)),
            pl.BlockSpec((2, RB, fprev), lambda i: (0, i, 0)),
            pl.BlockSpec((3, fi, fo), lambda i: (0, 0, 0)),
            pl.BlockSpec((1, fo), lambda i: (0, 0)),
        ]

    return pl.pallas_call(
        body,
        grid=(GR,),
        in_specs=in_specs,
        out_specs=[
            pl.BlockSpec((RB, fo), lambda i: (i, 0)),
            pl.BlockSpec((RB, fo), lambda i: (i, 0)),
            pl.BlockSpec((RB, f2), lambda i: (i, 0)),
        ],
        out_shape=[
            jax.ShapeDtypeStruct((NROWS, fo), jnp.float32),
            jax.ShapeDtypeStruct((NROWS, fo), jnp.float32),
            jax.ShapeDtypeStruct((NROWS, f2), jnp.float32),
        ],
    )


def _make_tcb(fo, f2, parts=NC):
    def body(y1_ref, z_ref, t_ref):
        y1 = y1_ref[...]
        if f2 > fo:
            y1 = jnp.pad(y1, ((0, 0), (0, f2 - fo)))
        t_ref[...] = y1 + 2.0 * jnp.sum(z_ref[...], axis=0)

    return pl.pallas_call(
        body,
        grid=(GR,),
        in_specs=[
            pl.BlockSpec((RB, fo), lambda i: (i, 0)),
            pl.BlockSpec((parts, RB, f2), lambda i: (0, i, 0)),
        ],
        out_specs=pl.BlockSpec((RB, f2), lambda i: (i, 0)),
        out_shape=jax.ShapeDtypeStruct((NROWS, f2), jnp.float32),
    )


def _final_body(yb_ref, u_ref, out_ref):
    logits = yb_ref[...] + jnp.sum(u_ref[...], axis=0)[:, :4]
    m = jnp.max(logits, axis=1, keepdims=True)
    z = logits - m
    lse = jnp.log(jnp.sum(jnp.exp(z), axis=1, keepdims=True))
    out_ref[...] = z - lse


def _tc_final(yb, u, f2, parts=NC):
    return pl.pallas_call(
        _final_body,
        grid=(GR,),
        in_specs=[
            pl.BlockSpec((RB, 4), lambda i: (i, 0)),
            pl.BlockSpec((parts, RB, f2), lambda i: (0, i, 0)),
        ],
        out_specs=pl.BlockSpec((RB, 4), lambda i: (i, 0)),
        out_shape=jax.ShapeDtypeStruct((NROWS, 4), jnp.float32),
    )(yb, u)


_FIS = (128, 64, 32, 16)
_FOS = (64, 32, 16, 4)
_F2S = (64, 32, 16, 4)


def kernel(x, edge_index, edge_attr, W1, b1, W2, b2, W3, b3, W4, b4):
    row = edge_index[0].reshape(NW, EW)
    col = edge_index[1].reshape(NW, EW)
    attr = edge_attr.reshape(NW, EW)

    npad = EWP - EW
    pad_idx = ((jnp.arange(npad, dtype=jnp.int32) * 97) % NODES)
    padf = jnp.broadcast_to(pad_idx, (NW, npad))
    row3 = jnp.concatenate([row, padf], axis=1).reshape(NW, NCH, CH)
    col3 = jnp.concatenate([col, padf], axis=1).reshape(NW, NCH, CH)
    attr3 = jnp.concatenate(
        [attr, jnp.zeros((NW, npad), jnp.float32)], axis=1
    ).reshape(NW, NCH, CH)

    zeros1 = jnp.zeros((NROWS,), jnp.float32)
    deg2 = _deg_kernel(row3, attr3, zeros1)
    dinv = _dinv_tc(deg2)
    w3 = _norm_kernel(row3, col3, attr3, dinv)

    xp = jnp.pad(x, ((0, NROWS - NODES), (0, 0)))
    ws = (W1, W2, W3, W4)
    bs = (b1, b2, b3, b4)

    yb = u = None
    for i in range(4):
        fi, fo, f2 = _FIS[i], _FOS[i], _F2S[i]
        tca = _make_tca(fi, fo, f2, first=(i == 0))
        if i == 0:
            yb, y1, y2 = tca(xp, ws[i], bs[i].reshape(1, fo))
        else:
            yb, y1, y2 = tca(yb, u, ws[i], bs[i].reshape(1, fo))
        if f2 == 4:
            parts = NW
            propf = lambda tb, *a: _prop4(tb.reshape(-1), *a).reshape(
                NW, NROWS, 4)
        else:
            parts = NC
            propf = _PROPS[f2]
        z = propf(y2, col3, row3, w3)
        t = _make_tcb(fo, f2, parts)(y1, z)
        u = propf(t, col3, row3, w3)

    out = _tc_final(yb, u, _F2S[3], parts=NW)
    return out[:NODES]


# speedup vs baseline: 1.2336x; 1.2336x over previous
import functools

import jax
import jax.numpy as jnp
from jax import lax
from jax.experimental import pallas as pl
from jax.experimental.pallas import tpu as pltpu
from jax.experimental.pallas import tpu_sc as plsc

NODES = 10000
EDGES = 320000
NROWS = 10240
NC, NS = 2, 16
NW = NC * NS
RZ = NROWS // NS
CH = 128
EW = EDGES // NW
NCH = 80
EWP = NCH * CH
RB = 1024
GR = NROWS // RB


def _mesh():
    return plsc.VectorSubcoreMesh(
        core_axis_name="c", subcore_axis_name="s", num_cores=NC, num_subcores=NS
    )


@functools.partial(
    pl.kernel,
    out_type=jax.ShapeDtypeStruct((NC, NROWS), jnp.float32),
    mesh=_mesh(),
    compiler_params=pltpu.CompilerParams(needs_layout_passes=False, use_tc_tiling_on_sc=False),
    scratch_types=[
        pltpu.VMEM((NCH, CH), jnp.int32),
        pltpu.VMEM((NCH, CH), jnp.float32),
        pltpu.VMEM((RZ,), jnp.float32),
        pltpu.VMEM_SHARED((NROWS,), jnp.float32),
    ],
)
def _deg_kernel(row_hbm, attr_hbm, out_hbm, row_v, attr_v, zbuf, acc):
    c = lax.axis_index("c")
    s = lax.axis_index("s")
    wid = s * NC + c
    pltpu.sync_copy(row_hbm.at[wid], row_v)
    pltpu.sync_copy(attr_hbm.at[wid], attr_v)

    zv = jnp.zeros((16,), jnp.float32)

    def zrow(r, carry):
        zbuf[pl.ds(r * 16, 16)] = zv
        return carry

    lax.fori_loop(0, RZ // 16, zrow, 0)
    pltpu.sync_copy(zbuf, acc.at[pl.ds(s * RZ, RZ)])
    plsc.subcore_barrier()

    def body(j, carry):
        pltpu.sync_copy(attr_v.at[j], acc.at[row_v.at[j]], add=True)
        return carry

    lax.fori_loop(0, NCH, body, 0)
    plsc.subcore_barrier()
    pltpu.sync_copy(acc.at[pl.ds(s * RZ, RZ)], out_hbm.at[c, pl.ds(s * RZ, RZ)])


@functools.partial(
    pl.kernel,
    out_type=jax.ShapeDtypeStruct((NW, NCH, CH), jnp.float32),
    mesh=_mesh(),
    compiler_params=pltpu.CompilerParams(needs_layout_passes=False, use_tc_tiling_on_sc=False),
    scratch_types=[
        pltpu.VMEM((NCH, CH), jnp.int32),
        pltpu.VMEM((NCH, CH), jnp.int32),
        pltpu.VMEM((NCH, CH), jnp.float32),
        pltpu.VMEM((NCH, CH), jnp.float32),
        pltpu.VMEM((NROWS,), jnp.float32),
    ],
)
def _norm_kernel(row_hbm, col_hbm, attr_hbm, dinv_hbm, out_hbm,
                 row_v, col_v, attr_v, w_v, dinv_v):
    c = lax.axis_index("c")
    s = lax.axis_index("s")
    wid = s * NC + c
    pltpu.sync_copy(row_hbm.at[wid], row_v)
    pltpu.sync_copy(col_hbm.at[wid], col_v)
    pltpu.sync_copy(attr_hbm.at[wid], attr_v)
    pltpu.sync_copy(dinv_hbm, dinv_v)

    def body(j, carry):
        for g in range(CH // 16):
            sl = pl.ds(g * 16, 16)
            r16 = row_v[j, sl]
            c16 = col_v[j, sl]
            a16 = attr_v[j, sl]
            dr = plsc.load_gather(dinv_v, [r16])
            dc = plsc.load_gather(dinv_v, [c16])
            w_v[j, sl] = -(dr * a16 * dc)
        return carry

    lax.fori_loop(0, NCH, body, 0)
    pltpu.sync_copy(w_v, out_hbm.at[wid])


def _make_prop(F, staged=False, NB=4):
    nq = F // 16

    @functools.partial(
        pl.kernel,
        out_type=jax.ShapeDtypeStruct((NC, NROWS, F), jnp.float32),
        mesh=_mesh(),
        compiler_params=pltpu.CompilerParams(needs_layout_passes=False, use_tc_tiling_on_sc=False),
        scratch_types=[
            pltpu.VMEM((NCH, CH), jnp.int32),
            pltpu.VMEM((NCH, CH), jnp.int32),
            pltpu.VMEM((NCH, CH), jnp.float32),
            pltpu.VMEM((NB, CH, F), jnp.float32),
            pltpu.VMEM((16, F), jnp.float32),
            pltpu.VMEM_SHARED((NROWS, F), jnp.float32),
        ] + ([pltpu.VMEM_SHARED((NROWS, F), jnp.float32)] if staged else [])
          + [pltpu.SemaphoreType.DMA] * (2 * NB),
    )
    def prop(table_hbm, col_hbm, row_hbm, w_hbm, out_hbm,
             col_v, row_v, w_v, rbuf, zbuf, acc, *rest):
        if staged:
            tbl_s = rest[0]
            sems = rest[1:]
        else:
            tbl_s = None
            sems = rest
        gsems = sems[:NB]
        ssems = sems[NB:]
        c = lax.axis_index("c")
        s = lax.axis_index("s")
        wid = s * NC + c
        pltpu.sync_copy(col_hbm.at[wid], col_v)
        pltpu.sync_copy(row_hbm.at[wid], row_v)
        pltpu.sync_copy(w_hbm.at[wid], w_v)
        if staged:
            pltpu.sync_copy(table_hbm.at[pl.ds(s * RZ, RZ)],
                            tbl_s.at[pl.ds(s * RZ, RZ)])

        zv = jnp.zeros((16,), jnp.float32)

        def zrow(r, carry):
            for q in range(nq):
                zbuf[r, pl.ds(q * 16, 16)] = zv
            return carry

        lax.fori_loop(0, 16, zrow, 0)
        for i in range(RZ // 16):
            pltpu.sync_copy(zbuf, acc.at[pl.ds(s * RZ + i * 16, 16)])
        plsc.subcore_barrier()

        gsrc = tbl_s if staged else table_hbm

        def g_start(j, b):
            pltpu.async_copy(gsrc.at[col_v.at[j]], rbuf.at[b], gsems[b])

        def g_wait(j, b):
            pltpu.make_async_copy(gsrc.at[col_v.at[j]], rbuf.at[b],
                                  gsems[b]).wait()

        def s_start(j, b):
            pltpu.async_copy(rbuf.at[b], acc.at[row_v.at[j]], ssems[b],
                             add=True)

        def s_wait(j, b):
            pltpu.make_async_copy(rbuf.at[b], acc.at[row_v.at[j]],
                                  ssems[b]).wait()

        def scale_chunk(j, b):
            def scale(g, c2):
                w16 = w_v[j, pl.ds(g * 16, 16)]
                for u in range(16):
                    e = g * 16 + u
                    we = w16[u]
                    for q in range(nq):
                        sl = pl.ds(q * 16, 16)
                        rbuf[b, e, sl] = rbuf[b, e, sl] * we
                return c2

            lax.fori_loop(0, CH // 16, scale, 0)

        g_start(0, 0)
        if NB == 4:
            g_start(1, 1)

            def quad(j4, carry):
                for b in range(NB):
                    j = j4 * NB + b
                    g_wait(j, b)
                    scale_chunk(j, b)
                    s_start(j, b)
                    bn = (b + 2) % NB

                    @pl.when(j >= 2)
                    def _():
                        s_wait(j - 2, bn)

                    @pl.when(j + 2 < NCH)
                    def _():
                        g_start(j + 2, bn)

                return carry

            lax.fori_loop(0, NCH // NB, quad, 0)
            s_wait(NCH - 2, (NCH - 2) % NB)
            s_wait(NCH - 1, (NCH - 1) % NB)
        else:
            def pair(j2, carry):
                for b in range(2):
                    j = j2 * 2 + b
                    g_wait(j, b)

                    @pl.when(j + 1 < NCH)
                    def _():
                        g_start(j + 1, 1 - b)

                    scale_chunk(j, b)
                    s_start(j, b)
                    s_wait(j, b)
                return carry

            lax.fori_loop(0, NCH // 2, pair, 0)
        plsc.subcore_barrier()
        pltpu.sync_copy(acc.at[pl.ds(s * RZ, RZ)],
                        out_hbm.at[c, pl.ds(s * RZ, RZ)])

    return prop


_PROPS = {64: _make_prop(64, staged=True, NB=2),
          32: _make_prop(32, staged=True, NB=4),
          16: _make_prop(16, staged=True, NB=4)}


@functools.partial(
    pl.kernel,
    out_type=jax.ShapeDtypeStruct((NW, NROWS * 4), jnp.float32),
    mesh=_mesh(),
    compiler_params=pltpu.CompilerParams(needs_layout_passes=False, use_tc_tiling_on_sc=False),
    scratch_types=[
        pltpu.VMEM((NCH, CH), jnp.int32),
        pltpu.VMEM((NCH, CH), jnp.int32),
        pltpu.VMEM((NCH, CH), jnp.float32),
        pltpu.VMEM((NROWS * 4,), jnp.float32),
        pltpu.VMEM((NROWS * 4,), jnp.float32),
    ],
)
def _prop4(table_hbm, col_hbm, row_hbm, w_hbm, out_hbm,
           col_v, row_v, w_v, tbl_v, acc_v):
    c = lax.axis_index("c")
    s = lax.axis_index("s")
    wid = s * NC + c
    pltpu.sync_copy(col_hbm.at[wid], col_v)
    pltpu.sync_copy(row_hbm.at[wid], row_v)
    pltpu.sync_copy(w_hbm.at[wid], w_v)
    pltpu.sync_copy(table_hbm, tbl_v)

    zv = jnp.zeros((16,), jnp.float32)

    def zrow(r, carry):
        for q in range(4):
            acc_v[pl.ds((r * 4 + q) * 16, 16)] = zv
        return carry

    lax.fori_loop(0, NROWS // 16, zrow, 0)

    def chunk(j, carry):
        for g in range(CH // 16):
            sl = pl.ds(g * 16, 16)
            col4 = col_v[j, sl] * 4
            row4 = row_v[j, sl] * 4
            w16 = w_v[j, sl]
            for f in range(4):
                vals = plsc.load_gather(tbl_v, [col4 + f]) * w16
                plsc.addupdate_scatter(acc_v, [row4 + f], vals)
        return carry

    lax.fori_loop(0, NCH, chunk, 0)
    pltpu.sync_copy(acc_v, out_hbm.at[wid])


def _dinv_body(d_ref, dinv_ref):
    d = d_ref[0] + d_ref[1]
    safe = jnp.where(d > 0, d, 1.0)
    dinv_ref[...] = jnp.where(d > 0, lax.rsqrt(safe), 0.0)


def _dinv_tc(deg2):
    return pl.pallas_call(
        _dinv_body,
        out_shape=jax.ShapeDtypeStruct((NROWS // 128, 128), jnp.float32),
    )(deg2.reshape(NC, NROWS // 128, 128)).reshape(NROWS)


def _make_tca(fi, fo, f2, first):
    def body(*refs):
        if first:
            x_ref, w_ref, b_ref, ybase_ref, y1_ref, y2_ref = refs
            xb = x_ref[...]
        else:
            yb_ref, u_ref, w_ref, b_ref, ybase_ref, y1_ref, y2_ref = refs
            xb = jax.nn.relu(
                yb_ref[...] + (u_ref[0] + u_ref[1])[:, :fi]
            )
        w = w_ref[...]
        ybase_ref[...] = (
            jnp.dot(xb, w[0] - w[2], preferred_element_type=jnp.float32)
            + b_ref[...]
        )
        y1_ref[...] = jnp.dot(xb, w[1], preferred_element_type=jnp.float32)
        y2 = jnp.dot(xb, w[2], preferred_element_type=jnp.float32)
        if f2 > fo:
            y2 = jnp.pad(y2, ((0, 0), (0, f2 - fo)))
        y2_ref[...] = y2

    if first:
        in_specs = [
            pl.BlockSpec((RB, fi), lambda i: (i, 0)),
            pl.BlockSpec((3, fi, fo), lambda i: (0, 0, 0)),
            pl.BlockSpec((1, fo), lambda i: (0, 0)),
        ]
    else:
        fprev = max(fi, 16)
        in_specs = [
            pl.BlockSpec((RB, fi), lambda i: (i, 0)),
            pl.BlockSpec((2, RB, fprev), lambda i: (0, i, 0)),
            pl.BlockSpec((3, fi, fo), lambda i: (0, 0, 0)),
            pl.BlockSpec((1, fo), lambda i: (0, 0)),
        ]

    return pl.pallas_call(
        body,
        grid=(GR,),
        in_specs=in_specs,
        out_specs=[
            pl.BlockSpec((RB, fo), lambda i: (i, 0)),
            pl.BlockSpec((RB, fo), lambda i: (i, 0)),
            pl.BlockSpec((RB, f2), lambda i: (i, 0)),
        ],
        out_shape=[
            jax.ShapeDtypeStruct((NROWS, fo), jnp.float32),
            jax.ShapeDtypeStruct((NROWS, fo), jnp.float32),
            jax.ShapeDtypeStruct((NROWS, f2), jnp.float32),
        ],
    )


def _make_tcb(fo, f2, parts=NC):
    def body(y1_ref, z_ref, t_ref):
        y1 = y1_ref[...]
        if f2 > fo:
            y1 = jnp.pad(y1, ((0, 0), (0, f2 - fo)))
        t_ref[...] = y1 + 2.0 * jnp.sum(z_ref[...], axis=0)

    return pl.pallas_call(
        body,
        grid=(GR,),
        in_specs=[
            pl.BlockSpec((RB, fo), lambda i: (i, 0)),
            pl.BlockSpec((parts, RB, f2), lambda i: (0, i, 0)),
        ],
        out_specs=pl.BlockSpec((RB, f2), lambda i: (i, 0)),
        out_shape=jax.ShapeDtypeStruct((NROWS, f2), jnp.float32),
    )


def _final_body(yb_ref, u_ref, out_ref):
    logits = yb_ref[...] + jnp.sum(u_ref[...], axis=0)[:, :4]
    m = jnp.max(logits, axis=1, keepdims=True)
    z = logits - m
    lse = jnp.log(jnp.sum(jnp.exp(z), axis=1, keepdims=True))
    out_ref[...] = z - lse


def _tc_final(yb, u, f2, parts=NC):
    return pl.pallas_call(
        _final_body,
        grid=(GR,),
        in_specs=[
            pl.BlockSpec((RB, 4), lambda i: (i, 0)),
            pl.BlockSpec((parts, RB, f2), lambda i: (0, i, 0)),
        ],
        out_specs=pl.BlockSpec((RB, 4), lambda i: (i, 0)),
        out_shape=jax.ShapeDtypeStruct((NROWS, 4), jnp.float32),
    )(yb, u)


_FIS = (128, 64, 32, 16)
_FOS = (64, 32, 16, 4)
_F2S = (64, 32, 16, 16)


def kernel(x, edge_index, edge_attr, W1, b1, W2, b2, W3, b3, W4, b4):
    row = edge_index[0].reshape(NW, EW)
    col = edge_index[1].reshape(NW, EW)
    attr = edge_attr.reshape(NW, EW)

    npad = EWP - EW
    pad_idx = ((jnp.arange(npad, dtype=jnp.int32) * 97) % NODES)
    padf = jnp.broadcast_to(pad_idx, (NW, npad))
    row3 = jnp.concatenate([row, padf], axis=1).reshape(NW, NCH, CH)
    col3 = jnp.concatenate([col, padf], axis=1).reshape(NW, NCH, CH)
    attr3 = jnp.concatenate(
        [attr, jnp.zeros((NW, npad), jnp.float32)], axis=1
    ).reshape(NW, NCH, CH)

    deg2 = _deg_kernel(row3, attr3)
    dinv = _dinv_tc(deg2)
    w3 = _norm_kernel(row3, col3, attr3, dinv)

    xp = jnp.pad(x, ((0, NROWS - NODES), (0, 0)))
    ws = (W1, W2, W3, W4)
    bs = (b1, b2, b3, b4)

    yb = u = None
    for i in range(4):
        fi, fo, f2 = _FIS[i], _FOS[i], _F2S[i]
        tca = _make_tca(fi, fo, f2, first=(i == 0))
        if i == 0:
            yb, y1, y2 = tca(xp, ws[i], bs[i].reshape(1, fo))
        else:
            yb, y1, y2 = tca(yb, u, ws[i], bs[i].reshape(1, fo))
        if f2 == 4:
            parts = NW
            propf = lambda tb, *a: _prop4(tb.reshape(-1), *a).reshape(
                NW, NROWS, 4)
        else:
            parts = NC
            propf = _PROPS[f2]
        z = propf(y2, col3, row3, w3)
        t = _make_tcb(fo, f2, parts)(y1, z)
        u = propf(t, col3, row3, w3)

    out = _tc_final(yb, u, _F2S[3], parts=NW if _F2S[3] == 4 else NC)
    return out[:NODES]


# speedup vs baseline: 1.3382x; 1.0848x over previous
import functools

import jax
import jax.numpy as jnp
from jax import lax
from jax.experimental import pallas as pl
from jax.experimental.pallas import tpu as pltpu
from jax.experimental.pallas import tpu_sc as plsc

NODES = 10000
EDGES = 320000
NROWS = 10240
NC, NS = 2, 16
NW = NC * NS
RZ = NROWS // NS
CH = 128
EW = EDGES // NW
NCH = 80
EWP = NCH * CH
RB = 1024
GR = NROWS // RB


def _mesh():
    return plsc.VectorSubcoreMesh(
        core_axis_name="c", subcore_axis_name="s", num_cores=NC, num_subcores=NS
    )


@functools.partial(
    pl.kernel,
    out_type=jax.ShapeDtypeStruct((NC, NROWS), jnp.float32),
    mesh=_mesh(),
    compiler_params=pltpu.CompilerParams(needs_layout_passes=False, use_tc_tiling_on_sc=False),
    scratch_types=[
        pltpu.VMEM((NCH, CH), jnp.int32),
        pltpu.VMEM((NCH, CH), jnp.float32),
        pltpu.VMEM((RZ,), jnp.float32),
        pltpu.VMEM_SHARED((NROWS,), jnp.float32),
    ],
)
def _deg_kernel(row_hbm, attr_hbm, out_hbm, row_v, attr_v, zbuf, acc):
    c = lax.axis_index("c")
    s = lax.axis_index("s")
    wid = s * NC + c
    pltpu.sync_copy(row_hbm.at[wid], row_v)
    pltpu.sync_copy(attr_hbm.at[wid], attr_v)

    zv = jnp.zeros((16,), jnp.float32)

    def zrow(r, carry):
        zbuf[pl.ds(r * 16, 16)] = zv
        return carry

    lax.fori_loop(0, RZ // 16, zrow, 0)
    pltpu.sync_copy(zbuf, acc.at[pl.ds(s * RZ, RZ)])
    plsc.subcore_barrier()

    def body(j, carry):
        pltpu.sync_copy(attr_v.at[j], acc.at[row_v.at[j]], add=True)
        return carry

    lax.fori_loop(0, NCH, body, 0)
    plsc.subcore_barrier()
    pltpu.sync_copy(acc.at[pl.ds(s * RZ, RZ)], out_hbm.at[c, pl.ds(s * RZ, RZ)])


@functools.partial(
    pl.kernel,
    out_type=jax.ShapeDtypeStruct((NW, NCH, CH), jnp.float32),
    mesh=_mesh(),
    compiler_params=pltpu.CompilerParams(needs_layout_passes=False, use_tc_tiling_on_sc=False),
    scratch_types=[
        pltpu.VMEM((NCH, CH), jnp.int32),
        pltpu.VMEM((NCH, CH), jnp.int32),
        pltpu.VMEM((NCH, CH), jnp.float32),
        pltpu.VMEM((NCH, CH), jnp.float32),
        pltpu.VMEM((NROWS,), jnp.float32),
    ],
)
def _norm_kernel(row_hbm, col_hbm, attr_hbm, dinv_hbm, out_hbm,
                 row_v, col_v, attr_v, w_v, dinv_v):
    c = lax.axis_index("c")
    s = lax.axis_index("s")
    wid = s * NC + c
    pltpu.sync_copy(row_hbm.at[wid], row_v)
    pltpu.sync_copy(col_hbm.at[wid], col_v)
    pltpu.sync_copy(attr_hbm.at[wid], attr_v)
    pltpu.sync_copy(dinv_hbm, dinv_v)

    def body(j, carry):
        for g in range(CH // 16):
            sl = pl.ds(g * 16, 16)
            r16 = row_v[j, sl]
            c16 = col_v[j, sl]
            a16 = attr_v[j, sl]
            dr = plsc.load_gather(dinv_v, [r16])
            dc = plsc.load_gather(dinv_v, [c16])
            w_v[j, sl] = -(dr * a16 * dc)
        return carry

    lax.fori_loop(0, NCH, body, 0)
    pltpu.sync_copy(w_v, out_hbm.at[wid])


def _make_prop(F, staged=False, NB=4):
    nq = F // 16

    @functools.partial(
        pl.kernel,
        out_type=jax.ShapeDtypeStruct((NC, NROWS, F), jnp.float32),
        mesh=_mesh(),
        compiler_params=pltpu.CompilerParams(needs_layout_passes=False, use_tc_tiling_on_sc=False),
        scratch_types=[
            pltpu.VMEM((NCH, CH), jnp.int32),
            pltpu.VMEM((NCH, CH), jnp.int32),
            pltpu.VMEM((NCH, CH), jnp.float32),
            pltpu.VMEM((NB, CH, F), jnp.float32),
            pltpu.VMEM((16, F), jnp.float32),
            pltpu.VMEM_SHARED((NROWS, F), jnp.float32),
        ] + ([pltpu.VMEM_SHARED((NROWS, F), jnp.float32)] if staged else [])
          + [pltpu.SemaphoreType.DMA] * (2 * NB),
    )
    def prop(table_hbm, col_hbm, row_hbm, w_hbm, out_hbm,
             col_v, row_v, w_v, rbuf, zbuf, acc, *rest):
        if staged:
            tbl_s = rest[0]
            sems = rest[1:]
        else:
            tbl_s = None
            sems = rest
        gsems = sems[:NB]
        ssems = sems[NB:]
        c = lax.axis_index("c")
        s = lax.axis_index("s")
        wid = s * NC + c
        pltpu.sync_copy(col_hbm.at[wid], col_v)
        pltpu.sync_copy(row_hbm.at[wid], row_v)
        pltpu.sync_copy(w_hbm.at[wid], w_v)
        if staged:
            pltpu.sync_copy(table_hbm.at[pl.ds(s * RZ, RZ)],
                            tbl_s.at[pl.ds(s * RZ, RZ)])

        zv = jnp.zeros((16,), jnp.float32)

        def zrow(r, carry):
            for q in range(nq):
                zbuf[r, pl.ds(q * 16, 16)] = zv
            return carry

        lax.fori_loop(0, 16, zrow, 0)
        for i in range(RZ // 16):
            pltpu.sync_copy(zbuf, acc.at[pl.ds(s * RZ + i * 16, 16)])
        plsc.subcore_barrier()

        gsrc = tbl_s if staged else table_hbm

        def g_start(j, b):
            pltpu.async_copy(gsrc.at[col_v.at[j]], rbuf.at[b], gsems[b])

        def g_wait(j, b):
            pltpu.make_async_copy(gsrc.at[col_v.at[j]], rbuf.at[b],
                                  gsems[b]).wait()

        def s_start(j, b):
            pltpu.async_copy(rbuf.at[b], acc.at[row_v.at[j]], ssems[b],
                             add=True)

        def s_wait(j, b):
            pltpu.make_async_copy(rbuf.at[b], acc.at[row_v.at[j]],
                                  ssems[b]).wait()

        def scale_chunk(j, b):
            def scale(g, c2):
                w16 = w_v[j, pl.ds(g * 16, 16)]
                for u in range(16):
                    e = g * 16 + u
                    we = w16[u]
                    for q in range(nq):
                        sl = pl.ds(q * 16, 16)
                        rbuf[b, e, sl] = rbuf[b, e, sl] * we
                return c2

            lax.fori_loop(0, CH // 16, scale, 0)

        g_start(0, 0)
        if NB == 4:
            g_start(1, 1)

            def quad(j4, carry):
                for b in range(NB):
                    j = j4 * NB + b
                    g_wait(j, b)
                    scale_chunk(j, b)
                    s_start(j, b)
                    bn = (b + 2) % NB

                    @pl.when(j >= 2)
                    def _():
                        s_wait(j - 2, bn)

                    @pl.when(j + 2 < NCH)
                    def _():
                        g_start(j + 2, bn)

                return carry

            lax.fori_loop(0, NCH // NB, quad, 0)
            s_wait(NCH - 2, (NCH - 2) % NB)
            s_wait(NCH - 1, (NCH - 1) % NB)
        else:
            def pair(j2, carry):
                for b in range(2):
                    j = j2 * 2 + b
                    g_wait(j, b)

                    @pl.when(j + 1 < NCH)
                    def _():
                        g_start(j + 1, 1 - b)

                    scale_chunk(j, b)
                    s_start(j, b)
                    s_wait(j, b)
                return carry

            lax.fori_loop(0, NCH // 2, pair, 0)
        plsc.subcore_barrier()
        pltpu.sync_copy(acc.at[pl.ds(s * RZ, RZ)],
                        out_hbm.at[c, pl.ds(s * RZ, RZ)])

    return prop


_PROPS = {F: _make_prop(F) for F in (64, 32, 16)}


def _dinv_body(d_ref, dinv_ref):
    d = d_ref[0] + d_ref[1]
    safe = jnp.where(d > 0, d, 1.0)
    dinv_ref[...] = jnp.where(d > 0, lax.rsqrt(safe), 0.0)


def _dinv_tc(deg2):
    return pl.pallas_call(
        _dinv_body,
        out_shape=jax.ShapeDtypeStruct((NROWS // 128, 128), jnp.float32),
    )(deg2.reshape(NC, NROWS // 128, 128)).reshape(NROWS)


def _make_tca(fi, fo, f2, first):
    def body(*refs):
        if first:
            x_ref, w_ref, b_ref, ybase_ref, y1_ref, y2_ref = refs
            xb = x_ref[...]
        else:
            yb_ref, u_ref, w_ref, b_ref, ybase_ref, y1_ref, y2_ref = refs
            xb = jax.nn.relu(
                yb_ref[...] + (u_ref[0] + u_ref[1])[:, :fi]
            )
        w = w_ref[...]
        ybase_ref[...] = (
            jnp.dot(xb, w[0] - w[2], preferred_element_type=jnp.float32)
            + b_ref[...]
        )
        y1_ref[...] = jnp.dot(xb, w[1], preferred_element_type=jnp.float32)
        y2 = jnp.dot(xb, w[2], preferred_element_type=jnp.float32)
        if f2 > fo:
            y2 = jnp.pad(y2, ((0, 0), (0, f2 - fo)))
        y2_ref[...] = y2

    if first:
        in_specs = [
            pl.BlockSpec((RB, fi), lambda i: (i, 0)),
            pl.BlockSpec((3, fi, fo), lambda i: (0, 0, 0)),
            pl.BlockSpec((1, fo), lambda i: (0, 0)),
        ]
    else:
        fprev = max(fi, 16)
        in_specs = [
            pl.BlockSpec((RB, fi), lambda i: (i, 0)),
            pl.BlockSpec((2, RB, fprev), lambda i: (0, i, 0)),
            pl.BlockSpec((3, fi, fo), lambda i: (0, 0, 0)),
            pl.BlockSpec((1, fo), lambda i: (0, 0)),
        ]

    return pl.pallas_call(
        body,
        grid=(GR,),
        in_specs=in_specs,
        out_specs=[
            pl.BlockSpec((RB, fo), lambda i: (i, 0)),
            pl.BlockSpec((RB, fo), lambda i: (i, 0)),
            pl.BlockSpec((RB, f2), lambda i: (i, 0)),
        ],
        out_shape=[
            jax.ShapeDtypeStruct((NROWS, fo), jnp.float32),
            jax.ShapeDtypeStruct((NROWS, fo), jnp.float32),
            jax.ShapeDtypeStruct((NROWS, f2), jnp.float32),
        ],
    )


def _make_tcb(fo, f2, parts=NC):
    def body(y1_ref, z_ref, t_ref):
        y1 = y1_ref[...]
        if f2 > fo:
            y1 = jnp.pad(y1, ((0, 0), (0, f2 - fo)))
        t_ref[...] = y1 + 2.0 * jnp.sum(z_ref[...], axis=0)

    return pl.pallas_call(
        body,
        grid=(GR,),
        in_specs=[
            pl.BlockSpec((RB, fo), lambda i: (i, 0)),
            pl.BlockSpec((parts, RB, f2), lambda i: (0, i, 0)),
        ],
        out_specs=pl.BlockSpec((RB, f2), lambda i: (i, 0)),
        out_shape=jax.ShapeDtypeStruct((NROWS, f2), jnp.float32),
    )


def _final_body(yb_ref, u_ref, out_ref):
    logits = yb_ref[...] + jnp.sum(u_ref[...], axis=0)[:, :4]
    m = jnp.max(logits, axis=1, keepdims=True)
    z = logits - m
    lse = jnp.log(jnp.sum(jnp.exp(z), axis=1, keepdims=True))
    out_ref[...] = z - lse


def _tc_final(yb, u, f2, parts=NC):
    return pl.pallas_call(
        _final_body,
        grid=(GR,),
        in_specs=[
            pl.BlockSpec((RB, 4), lambda i: (i, 0)),
            pl.BlockSpec((parts, RB, f2), lambda i: (0, i, 0)),
        ],
        out_specs=pl.BlockSpec((RB, 4), lambda i: (i, 0)),
        out_shape=jax.ShapeDtypeStruct((NROWS, 4), jnp.float32),
    )(yb, u)


_FIS = (128, 64, 32, 16)
_FOS = (64, 32, 16, 4)
_F2S = (64, 32, 16, 16)


def kernel(x, edge_index, edge_attr, W1, b1, W2, b2, W3, b3, W4, b4):
    row = edge_index[0].reshape(NW, EW)
    col = edge_index[1].reshape(NW, EW)
    attr = edge_attr.reshape(NW, EW)

    npad = EWP - EW
    pad_idx = ((jnp.arange(npad, dtype=jnp.int32) * 97) % NODES)
    padf = jnp.broadcast_to(pad_idx, (NW, npad))
    row3 = jnp.concatenate([row, padf], axis=1).reshape(NW, NCH, CH)
    col3 = jnp.concatenate([col, padf], axis=1).reshape(NW, NCH, CH)
    attr3 = jnp.concatenate(
        [attr, jnp.zeros((NW, npad), jnp.float32)], axis=1
    ).reshape(NW, NCH, CH)

    deg2 = _deg_kernel(row3, attr3)
    dinv = _dinv_tc(deg2)
    w3 = _norm_kernel(row3, col3, attr3, dinv)

    xp = jnp.pad(x, ((0, NROWS - NODES), (0, 0)))
    ws = (W1, W2, W3, W4)
    bs = (b1, b2, b3, b4)

    yb = u = None
    for i in range(4):
        fi, fo, f2 = _FIS[i], _FOS[i], _F2S[i]
        tca = _make_tca(fi, fo, f2, first=(i == 0))
        if i == 0:
            yb, y1, y2 = tca(xp, ws[i], bs[i].reshape(1, fo))
        else:
            yb, y1, y2 = tca(yb, u, ws[i], bs[i].reshape(1, fo))
        z = _PROPS[f2](y2, col3, row3, w3)
        t = _make_tcb(fo, f2)(y1, z)
        u = _PROPS[f2](t, col3, row3, w3)

    out = _tc_final(yb, u, _F2S[3])
    return out[:NODES]
